# Initial kernel scaffold; baseline (speedup 1.0000x reference)
import jax
import jax.numpy as jnp
from jax.experimental import pallas as pl


def kernel(x_user, x_item, edge_index_ui, edge_index_iu, params):
    raise NotImplementedError("write your pallas kernel here")


# speedup vs baseline: 4.8498x; 4.8498x over previous
import functools

import jax
import jax.numpy as jnp
from jax import lax
from jax.experimental import pallas as pl
from jax.experimental.pallas import tpu as pltpu
from jax.experimental.pallas import tpu_sc as plsc

N = 10000
D = 128
NPAD = 10240
RPT = NPAD // 16
E = 320000
CHUNK = 128
CH = 157
EPT = CH * CHUNK
EPAD = 16 * EPT
SINK = N
BLK = 1000


def _sc_segsum(table, src2, dst2):
    mesh = plsc.VectorSubcoreMesh(core_axis_name="c", subcore_axis_name="s")

    @functools.partial(
        pl.kernel,
        mesh=mesh,
        out_type=(
            jax.ShapeDtypeStruct((2, NPAD, D), jnp.float32),
            jax.ShapeDtypeStruct((2, NPAD), jnp.float32),
        ),
        scratch_types=[
            pltpu.VMEM((CHUNK, D), jnp.float32),
            pltpu.VMEM((CHUNK,), jnp.int32),
            pltpu.VMEM((CHUNK,), jnp.int32),
            pltpu.VMEM((CHUNK,), jnp.float32),
            pltpu.VMEM_SHARED((NPAD, D), jnp.float32),
            pltpu.VMEM_SHARED((NPAD,), jnp.float32),
            pltpu.SemaphoreType.DMA,
        ],
    )
    def k(table_h, src_h, dst_h, s_h, cnt_h,
          rows_v, sidx_v, didx_v, ones_v, acc_s, cnt_s, sem):
        cid = lax.axis_index("c")
        sid = lax.axis_index("s")
        for j in range(8):
            ones_v[pl.ds(16 * j, 16)] = jnp.ones((16,), jnp.float32)

        def _zrow(i, c):
            for j in range(8):
                rows_v[i, pl.ds(16 * j, 16)] = jnp.zeros((16,), jnp.float32)
            return c

        lax.fori_loop(0, CHUNK, _zrow, 0)

        base_r = sid * RPT
        for z in range(RPT // CHUNK):
            pltpu.sync_copy(rows_v, acc_s.at[pl.ds(base_r + z * CHUNK, CHUNK)])
            pltpu.sync_copy(rows_v.at[z], cnt_s.at[pl.ds(base_r + z * CHUNK, CHUNK)])
        plsc.subcore_barrier()

        ebase = sid * EPT

        def _body(c, carry):
            off = ebase + c * CHUNK
            pltpu.sync_copy(src_h.at[cid, pl.ds(off, CHUNK)], sidx_v)
            pltpu.sync_copy(dst_h.at[cid, pl.ds(off, CHUNK)], didx_v)
            pltpu.async_copy(table_h.at[sidx_v], rows_v, sem).wait()
            pltpu.sync_copy(ones_v, cnt_s.at[didx_v], add=True)
            pltpu.sync_copy(rows_v, acc_s.at[didx_v], add=True)
            return carry

        lax.fori_loop(0, CH, _body, 0)
        plsc.subcore_barrier()
        pltpu.sync_copy(acc_s.at[pl.ds(base_r, RPT)], s_h.at[cid, pl.ds(base_r, RPT)])
        pltpu.sync_copy(cnt_s.at[pl.ds(base_r, RPT)], cnt_h.at[cid, pl.ds(base_r, RPT)])

    return k(table, src2, dst2)


def _tc_body(s_ref, c_ref, x_ref, wl_ref, wr_ref, wlsl_ref, wrsl_ref,
             b_ref, bsl_ref, wlin_ref, blin_ref, o_ref, *, final):
    inv = 1.0 / jnp.maximum(c_ref[0], 1.0)
    agg = s_ref[0] * inv
    wc = wr_ref[0] + wlsl_ref[0] + wrsl_ref[0]
    h = (jnp.dot(agg, wl_ref[0], preferred_element_type=jnp.float32)
         + jnp.dot(x_ref[...], wc, preferred_element_type=jnp.float32)
         + b_ref[0] + bsl_ref[0])
    h = jnp.maximum(h, 0.0)
    if final:
        h = jnp.dot(h, wlin_ref[...], preferred_element_type=jnp.float32) + blin_ref[...]
    o_ref[...] = h


def _tc_layer(s, cnt3, x, wl, wr, wlsl, wrsl, b, bsl, wlin, blin, final):
    nb = N // BLK
    grid = (2, nb)
    return pl.pallas_call(
        functools.partial(_tc_body, final=final),
        grid=grid,
        in_specs=[
            pl.BlockSpec((1, BLK, D), lambda t, i: (t, i, 0)),
            pl.BlockSpec((1, BLK, 1), lambda t, i: (t, i, 0)),
            pl.BlockSpec((BLK, D), lambda t, i: (t * nb + i, 0)),
            pl.BlockSpec((1, D, D), lambda t, i: (t, 0, 0)),
            pl.BlockSpec((1, D, D), lambda t, i: (t, 0, 0)),
            pl.BlockSpec((1, D, D), lambda t, i: (t, 0, 0)),
            pl.BlockSpec((1, D, D), lambda t, i: (t, 0, 0)),
            pl.BlockSpec((1, 1, D), lambda t, i: (t, 0, 0)),
            pl.BlockSpec((1, 1, D), lambda t, i: (t, 0, 0)),
            pl.BlockSpec((D, D), lambda t, i: (0, 0)),
            pl.BlockSpec((1, D), lambda t, i: (0, 0)),
        ],
        out_specs=pl.BlockSpec((BLK, D), lambda t, i: (t * nb + i, 0)),
        out_shape=jax.ShapeDtypeStruct((2 * N, D), jnp.float32),
    )(s, cnt3, x, wl, wr, wlsl, wrsl, b, bsl, wlin, blin)


def kernel(x_user, x_item, edge_index_ui, edge_index_iu, params):
    src_iu = edge_index_iu[0].astype(jnp.int32) + N
    dst_iu = edge_index_iu[1].astype(jnp.int32)
    src_ui = edge_index_ui[0].astype(jnp.int32)
    dst_ui = edge_index_ui[1].astype(jnp.int32)
    npad_e = EPAD - E
    zpad = jnp.zeros((npad_e,), jnp.int32)
    spad = jnp.full((npad_e,), SINK, jnp.int32)
    src2 = jnp.stack([jnp.concatenate([src_iu, zpad]),
                      jnp.concatenate([src_ui, zpad])])
    dst2 = jnp.stack([jnp.concatenate([dst_iu, spad]),
                      jnp.concatenate([dst_ui, spad])])

    x = jnp.concatenate([x_user, x_item], axis=0)
    for l in range(2):
        s, cnt = _sc_segsum(x, src2, dst2)
        wl = jnp.stack([params['Wl_%d_iu' % l], params['Wl_%d_ui' % l]])
        wr = jnp.stack([params['Wr_%d_iu' % l], params['Wr_%d_ui' % l]])
        wlsl = jnp.stack([params['Wl_%d_sl_u' % l], params['Wl_%d_sl_i' % l]])
        wrsl = jnp.stack([params['Wr_%d_sl_u' % l], params['Wr_%d_sl_i' % l]])
        b = jnp.stack([params['bl_%d_iu' % l], params['bl_%d_ui' % l]]).reshape(2, 1, D)
        bsl = jnp.stack([params['bl_%d_sl_u' % l], params['bl_%d_sl_i' % l]]).reshape(2, 1, D)
        x = _tc_layer(s, cnt.reshape(2, NPAD, 1), x, wl, wr, wlsl, wrsl,
                      b, bsl, params['W_lin'], params['b_lin'].reshape(1, D),
                      final=(l == 1))
    return x[:N], x[N:]
